# Initial kernel scaffold; baseline (speedup 1.0000x reference)
import jax
import jax.numpy as jnp
from jax.experimental import pallas as pl


def kernel(x_adresse, x_batiment, x_parcelle, ei_acces, ei_desservi, ei_appartient, ei_contient, ei_spat_bat, ei_spat_par, ei_spat_adr, ea_acces, ea_desservi, ea_appartient, ea_contient, Wl, Wr, att_w, bias_w, We, lin_W, lin_b):
    raise NotImplementedError("write your pallas kernel here")


# speedup vs baseline: 1.9246x; 1.9246x over previous
import functools

import jax
import jax.numpy as jnp
from jax import lax
from jax.experimental import pallas as pl
from jax.experimental.pallas import tpu as pltpu
from jax.experimental.pallas import tpu_sc as plsc

N = 10000
E = 160000
F = 128
H = 128
OUT = 64
ED = 16
RELS = [("adr", "bat", 0), ("bat", "adr", 1), ("bat", "par", 2), ("par", "bat", 3), ("bat", "bat", None), ("par", "par", None), ("adr", "adr", None)]
TYPES = ["adr", "bat", "par"]

NC, NS, L = 2, 16, 16
NW = NC * NS
E_PAD = 163840
PER_W = E_PAD // NW
CHUNK = 512
N_CHUNKS = PER_W // CHUNK


def _gather_body(table_hbm, idx_hbm, out_hbm, idx_v, rows_v, sem):
    wid = lax.axis_index("s") * NC + lax.axis_index("c")
    base0 = wid * PER_W

    def step(j, _):
        base = base0 + j * CHUNK
        pltpu.sync_copy(idx_hbm.at[pl.ds(base, CHUNK)], idx_v)
        pltpu.async_copy(table_hbm.at[idx_v], rows_v, sem).wait()
        pltpu.sync_copy(rows_v, out_hbm.at[pl.ds(base, CHUNK)])
        return 0

    lax.fori_loop(0, N_CHUNKS, step, 0)


@jax.jit
def _sc_gather(table, idx):
    mesh = plsc.VectorSubcoreMesh(core_axis_name="c", subcore_axis_name="s")
    k = functools.partial(
        pl.kernel,
        mesh=mesh,
        out_type=jax.ShapeDtypeStruct((E_PAD, F), jnp.float32),
        scratch_types=[
            pltpu.VMEM((CHUNK,), jnp.int32),
            pltpu.VMEM((CHUNK, F), jnp.float32),
            pltpu.SemaphoreType.DMA,
        ],
    )(_gather_body)
    return k(table, idx)


def _pad_idx(ix):
    return jnp.concatenate([ix, jnp.zeros((E_PAD - E,), jnp.int32)])


def _gatv2(x_src, x_dst, ei, Wl_r, Wr_r, aw, b, ea, We_r):
    xl = x_src @ Wl_r
    xr = x_dst @ Wr_r
    src = ei[0]
    dst = ei[1]
    a_rows = _sc_gather(xl, _pad_idx(src))[:E]
    b_rows = _sc_gather(xr, _pad_idx(dst))[:E]
    m = a_rows + b_rows
    if ea is not None:
        m = m + ea @ We_r
    e = jax.nn.leaky_relu(m, 0.2) @ aw
    ex = jnp.exp(e)
    den = jax.ops.segment_sum(ex, dst, num_segments=N)
    alpha = ex / (den[dst] + 1e-16)
    out = jax.ops.segment_sum(alpha[:, None] * a_rows, dst, num_segments=N)
    return out + b


def kernel(x_adresse, x_batiment, x_parcelle, ei_acces, ei_desservi, ei_appartient, ei_contient, ei_spat_bat, ei_spat_par, ei_spat_adr, ea_acces, ea_desservi, ea_appartient, ea_contient, Wl, Wr, att_w, bias_w, We, lin_W, lin_b):
    eis = [ei_acces, ei_desservi, ei_appartient, ei_contient, ei_spat_bat, ei_spat_par, ei_spat_adr]
    eas = [ea_acces, ea_desservi, ea_appartient, ea_contient]
    xs = {"adr": x_adresse, "bat": x_batiment, "par": x_parcelle}
    for l in range(2):
        new = {t: jnp.zeros((N, H), dtype=jnp.float32) for t in TYPES}
        for r, (s, d, ai) in enumerate(RELS):
            ea = eas[ai] if ai is not None else None
            We_r = We[l, ai] if ai is not None else None
            o = _gatv2(xs[s], xs[d], eis[r], Wl[l, r], Wr[l, r], att_w[l, r], bias_w[l, r], ea, We_r)
            new[d] = new[d] + o
        xs = {t: jax.nn.relu(v) for t, v in new.items()}
    outs = [xs[t] @ lin_W[i] + lin_b[i] for i, t in enumerate(TYPES)]
    return jnp.stack(outs)


# speedup vs baseline: 3.0054x; 1.5616x over previous
import jax
import jax.numpy as jnp
from jax import lax
from jax.experimental import pallas as pl
from jax.experimental.pallas import tpu as pltpu
from jax.experimental.pallas import tpu_sc as plsc

N = 10000
E = 160000
F = 128
H = 128
OUT = 64
ED = 16
RELS = [("adr", "bat", 0), ("bat", "adr", 1), ("bat", "par", 2), ("par", "bat", 3), ("bat", "bat", None), ("par", "par", None), ("adr", "adr", None)]
TYPES = ["adr", "bat", "par"]

NC, NS, L = 2, 16, 16
NW = NC * NS
E_PAD = 163840
PER_W = E_PAD // NW
CHUNK = 64
N_CHUNKS = PER_W // CHUNK
NP = 10112
STRIPE = NP // NS
KG = H // L


def _lane_sum_splat(v):
    dn = lax.GatherDimensionNumbers(
        offset_dims=(), collapsed_slice_dims=(0,), start_index_map=(0,))
    iota = lax.iota(jnp.int32, 16)
    for sh in (8, 4, 2, 1):
        idx = (iota + sh) & 15
        rot = lax.gather(v, idx[:, None], dn, slice_sizes=(1,),
                         mode=lax.GatherScatterMode.PROMISE_IN_BOUNDS)
        v = v + rot
    return v


def _edge_body(has_ce, refs):
    if has_ce:
        (xl_hbm, xr_hbm, ce_hbm, src_hbm, dstg_hbm, dsts_hbm, aw_hbm, zr_hbm,
         acc_out, ex_out,
         src_v, dstg_v, dsts_v, a_v, b_v, c_v, ex_v, aw_v, acc_s, sem) = refs
    else:
        (xl_hbm, xr_hbm, src_hbm, dstg_hbm, dsts_hbm, aw_hbm, zr_hbm,
         acc_out, ex_out,
         src_v, dstg_v, dsts_v, a_v, b_v, ex_v, aw_v, acc_s, sem) = refs
        c_v = None
    cid = lax.axis_index("c")
    sid = lax.axis_index("s")
    wid = sid * NC + cid
    base0 = wid * PER_W
    iota = lax.iota(jnp.int32, 16)
    zero16 = jnp.zeros((16,), jnp.float32)

    pltpu.sync_copy(zr_hbm, acc_s.at[pl.ds(sid * STRIPE, STRIPE)])
    plsc.subcore_barrier()

    pltpu.sync_copy(aw_hbm, aw_v)
    awk = [aw_v[pl.ds(16 * k, 16)] for k in range(KG)]

    def chunk_step(j, _):
        base = base0 + j * CHUNK
        pltpu.sync_copy(src_hbm.at[pl.ds(base, CHUNK)], src_v)
        pltpu.sync_copy(dstg_hbm.at[pl.ds(base, CHUNK)], dstg_v)
        pltpu.sync_copy(dsts_hbm.at[pl.ds(base, CHUNK)], dsts_v)
        pltpu.async_copy(xl_hbm.at[src_v], a_v, sem).wait()
        pltpu.async_copy(xr_hbm.at[dstg_v], b_v, sem).wait()
        if has_ce:
            pltpu.sync_copy(ce_hbm.at[pl.ds(base, CHUNK)], c_v)

        def edge_step(e, _):
            aks = []
            acc = zero16
            for k in range(KG):
                ak = a_v[e, pl.ds(16 * k, 16)]
                aks.append(ak)
                m = ak + b_v[e, pl.ds(16 * k, 16)]
                if has_ce:
                    m = m + c_v[e, pl.ds(16 * k, 16)]
                lr = jnp.maximum(m, 0.2 * m)
                acc = acc + lr * awk[k]
            exv = jnp.exp(_lane_sum_splat(acc))
            for k in range(KG):
                a_v[e, pl.ds(16 * k, 16)] = aks[k] * exv
            e16 = (e >> 4) << 4
            win = ex_v[pl.ds(e16, 16)]
            ex_v[pl.ds(e16, 16)] = jnp.where(iota == (e & 15), exv, win)
            return 0

        lax.fori_loop(0, CHUNK, edge_step, 0)
        pltpu.sync_copy(a_v, acc_s.at[dsts_v], add=True)
        pltpu.sync_copy(ex_v, ex_out.at[pl.ds(base, CHUNK)])
        return 0

    lax.fori_loop(0, N_CHUNKS, chunk_step, 0)
    plsc.subcore_barrier()
    pltpu.sync_copy(acc_s.at[pl.ds(sid * STRIPE, STRIPE)],
                    acc_out.at[pl.ds(cid * NP + sid * STRIPE, STRIPE)])


def _make_edge_kernel(has_ce):
    mesh = plsc.VectorSubcoreMesh(core_axis_name="c", subcore_axis_name="s")
    scratch = [
        pltpu.VMEM((CHUNK,), jnp.int32),
        pltpu.VMEM((CHUNK,), jnp.int32),
        pltpu.VMEM((CHUNK,), jnp.int32),
        pltpu.VMEM((CHUNK, F), jnp.float32),
        pltpu.VMEM((CHUNK, F), jnp.float32),
    ]
    if has_ce:
        scratch.append(pltpu.VMEM((CHUNK, F), jnp.float32))
    scratch += [
        pltpu.VMEM((CHUNK,), jnp.float32),
        pltpu.VMEM((F,), jnp.float32),
        pltpu.VMEM_SHARED((NP, F), jnp.float32),
        pltpu.SemaphoreType.DMA,
    ]

    def body(*refs):
        _edge_body(has_ce, refs)

    return pl.kernel(
        body,
        mesh=mesh,
        out_type=(
            jax.ShapeDtypeStruct((NC * NP, F), jnp.float32),
            jax.ShapeDtypeStruct((E_PAD,), jnp.float32),
        ),
        scratch_types=scratch,
    )


def _edge_pass(xl, xr, ce, srcp, dstgp, dstsp, aw, dst):
    zr = jnp.zeros((STRIPE, F), jnp.float32)
    k = _make_edge_kernel(ce is not None)
    if ce is not None:
        acc, ex = k(xl, xr, ce, srcp, dstgp, dstsp, aw, zr)
    else:
        acc, ex = k(xl, xr, srcp, dstgp, dstsp, aw, zr)
    acc = acc.reshape(NC, NP, F)
    num = acc[0, :N] + acc[1, :N]
    d = jax.ops.segment_sum(ex[:E], dst, num_segments=N)
    return num / (d + 1e-16)[:, None]


def _pad_e(ix, fill):
    return jnp.concatenate([ix, jnp.full((E_PAD - E,), fill, jnp.int32)])


def kernel(x_adresse, x_batiment, x_parcelle, ei_acces, ei_desservi, ei_appartient, ei_contient, ei_spat_bat, ei_spat_par, ei_spat_adr, ea_acces, ea_desservi, ea_appartient, ea_contient, Wl, Wr, att_w, bias_w, We, lin_W, lin_b):
    eis = [ei_acces, ei_desservi, ei_appartient, ei_contient, ei_spat_bat, ei_spat_par, ei_spat_adr]
    eas = [ea_acces, ea_desservi, ea_appartient, ea_contient]
    srcs = [_pad_e(ei[0], 0) for ei in eis]
    dstgs = [_pad_e(ei[1], 0) for ei in eis]
    dstss = [_pad_e(ei[1], N) for ei in eis]

    xs = {"adr": x_adresse, "bat": x_batiment, "par": x_parcelle}
    for l in range(2):
        new = {t: jnp.zeros((N, H), dtype=jnp.float32) for t in TYPES}
        for r, (s, d, ai) in enumerate(RELS):
            xl = xs[s] @ Wl[l, r]
            xr = xs[d] @ Wr[l, r]
            if ai is not None:
                ce = eas[ai] @ We[l, ai]
                ce = jnp.concatenate([ce, jnp.zeros((E_PAD - E, F), jnp.float32)])
            else:
                ce = None
            o = _edge_pass(xl, xr, ce, srcs[r], dstgs[r], dstss[r], att_w[l, r], eis[r][1])
            new[d] = new[d] + o + bias_w[l, r]
        xs = {t: jax.nn.relu(v) for t, v in new.items()}
    outs = [xs[t] @ lin_W[i] + lin_b[i] for i, t in enumerate(TYPES)]
    return jnp.stack(outs)


# speedup vs baseline: 4.4014x; 1.4645x over previous
import jax
import jax.numpy as jnp
from jax import lax
from jax.experimental import pallas as pl
from jax.experimental.pallas import tpu as pltpu
from jax.experimental.pallas import tpu_sc as plsc

N = 10000
E = 160000
F = 128
H = 128
OUT = 64
ED = 16
RELS = [("adr", "bat", 0), ("bat", "adr", 1), ("bat", "par", 2), ("par", "bat", 3), ("bat", "bat", None), ("par", "par", None), ("adr", "adr", None)]
TYPES = ["adr", "bat", "par"]

NC, NS, L = 2, 16, 16
NW = NC * NS
E_PAD = 163840
PER_W = E_PAD // NW
CHUNK = 80
N_CHUNKS = PER_W // CHUNK
NP = 10112
STRIPE = NP // NS
KG = H // L


def _lane_sum_splat(v):
    dn = lax.GatherDimensionNumbers(
        offset_dims=(), collapsed_slice_dims=(0,), start_index_map=(0,))
    iota = lax.iota(jnp.int32, 16)
    for sh in (8, 4, 2, 1):
        idx = (iota + sh) & 15
        rot = lax.gather(v, idx[:, None], dn, slice_sizes=(1,),
                         mode=lax.GatherScatterMode.PROMISE_IN_BOUNDS)
        v = v + rot
    return v


def _edge_body(has_ce, refs):
    if has_ce:
        (xl_hbm, xr_hbm, ce_hbm, src_hbm, dstg_hbm, dsts_hbm, aw_hbm, zr_hbm,
         acc_out, ex_out,
         src_v, dstg_v, dsts_v, a_v, b_v, c_v, ex_v, aw_v, acc_s,
         sem_i1, sem_i2, sem_i3, sem_g1, sem_g2, sem_c, sem_s1, sem_s2) = refs
    else:
        (xl_hbm, xr_hbm, src_hbm, dstg_hbm, dsts_hbm, aw_hbm, zr_hbm,
         acc_out, ex_out,
         src_v, dstg_v, dsts_v, a_v, b_v, ex_v, aw_v, acc_s,
         sem_i1, sem_i2, sem_i3, sem_g1, sem_g2, sem_c, sem_s1, sem_s2) = refs
        c_v = None
    cid = lax.axis_index("c")
    sid = lax.axis_index("s")
    wid = sid * NC + cid
    base0 = wid * PER_W
    iota = lax.iota(jnp.int32, 16)
    zero16 = jnp.zeros((16,), jnp.float32)

    pltpu.sync_copy(zr_hbm, acc_s.at[pl.ds(sid * STRIPE, STRIPE)])
    plsc.subcore_barrier()

    pltpu.sync_copy(aw_hbm, aw_v)
    awk = [aw_v[pl.ds(16 * k, 16)] for k in range(KG)]

    def chunk_step(j, _):
        base = base0 + j * CHUNK
        di = pltpu.async_copy(src_hbm.at[pl.ds(base, CHUNK)], src_v, sem_i1)
        di2 = pltpu.async_copy(dstg_hbm.at[pl.ds(base, CHUNK)], dstg_v, sem_i2)
        di3 = pltpu.async_copy(dsts_hbm.at[pl.ds(base, CHUNK)], dsts_v, sem_i3)
        if has_ce:
            dc = pltpu.async_copy(ce_hbm.at[pl.ds(base, CHUNK)], c_v, sem_c)
        di.wait()
        di2.wait()
        di3.wait()
        dg = pltpu.async_copy(xl_hbm.at[src_v], a_v, sem_g1)
        dg2 = pltpu.async_copy(xr_hbm.at[dstg_v], b_v, sem_g2)
        dg.wait()
        dg2.wait()
        if has_ce:
            dc.wait()

        def edge_step(e, _):
            aks = []
            acc = zero16
            for k in range(KG):
                ak = a_v[e, pl.ds(16 * k, 16)]
                aks.append(ak)
                m = ak + b_v[e, pl.ds(16 * k, 16)]
                if has_ce:
                    m = m + c_v[e, pl.ds(16 * k, 16)]
                lr = jnp.maximum(m, 0.2 * m)
                acc = acc + lr * awk[k]
            exv = jnp.exp(_lane_sum_splat(acc))
            for k in range(KG):
                a_v[e, pl.ds(16 * k, 16)] = aks[k] * exv
            e16 = (e >> 4) << 4
            win = ex_v[pl.ds(e16, 16)]
            ex_v[pl.ds(e16, 16)] = jnp.where(iota == (e & 15), exv, win)
            return 0

        lax.fori_loop(0, CHUNK, edge_step, 0)
        ds1 = pltpu.async_copy(a_v, acc_s.at[dsts_v], sem_s1, add=True)
        ds2 = pltpu.async_copy(ex_v, ex_out.at[pl.ds(base, CHUNK)], sem_s2)
        ds1.wait()
        ds2.wait()
        return 0

    lax.fori_loop(0, N_CHUNKS, chunk_step, 0)
    plsc.subcore_barrier()
    pltpu.sync_copy(acc_s.at[pl.ds(sid * STRIPE, STRIPE)],
                    acc_out.at[pl.ds(cid * NP + sid * STRIPE, STRIPE)])


def _make_edge_kernel(has_ce):
    mesh = plsc.VectorSubcoreMesh(core_axis_name="c", subcore_axis_name="s")
    scratch = [
        pltpu.VMEM((CHUNK,), jnp.int32),
        pltpu.VMEM((CHUNK,), jnp.int32),
        pltpu.VMEM((CHUNK,), jnp.int32),
        pltpu.VMEM((CHUNK, F), jnp.float32),
        pltpu.VMEM((CHUNK, F), jnp.float32),
    ]
    if has_ce:
        scratch.append(pltpu.VMEM((CHUNK, F), jnp.float32))
    scratch += [
        pltpu.VMEM((CHUNK,), jnp.float32),
        pltpu.VMEM((F,), jnp.float32),
        pltpu.VMEM_SHARED((NP, F), jnp.float32),
    ] + [pltpu.SemaphoreType.DMA] * 8

    def body(*refs):
        _edge_body(has_ce, refs)

    return pl.kernel(
        body,
        mesh=mesh,
        out_type=(
            jax.ShapeDtypeStruct((NC * NP, F), jnp.float32),
            jax.ShapeDtypeStruct((E_PAD,), jnp.float32),
        ),
        scratch_types=scratch,
    )


def _edge_pass(xl, xr, ce, srcp, dstgp, dstsp, aw, dst):
    zr = jnp.zeros((STRIPE, F), jnp.float32)
    k = _make_edge_kernel(ce is not None)
    if ce is not None:
        acc, ex = k(xl, xr, ce, srcp, dstgp, dstsp, aw, zr)
    else:
        acc, ex = k(xl, xr, srcp, dstgp, dstsp, aw, zr)
    acc = acc.reshape(NC, NP, F)
    num = acc[0, :N] + acc[1, :N]
    d = jax.ops.segment_sum(ex[:E], dst, num_segments=N)
    return num / (d + 1e-16)[:, None]


def _pad_e(ix, fill):
    return jnp.concatenate([ix, jnp.full((E_PAD - E,), fill, jnp.int32)])


def kernel(x_adresse, x_batiment, x_parcelle, ei_acces, ei_desservi, ei_appartient, ei_contient, ei_spat_bat, ei_spat_par, ei_spat_adr, ea_acces, ea_desservi, ea_appartient, ea_contient, Wl, Wr, att_w, bias_w, We, lin_W, lin_b):
    eis = [ei_acces, ei_desservi, ei_appartient, ei_contient, ei_spat_bat, ei_spat_par, ei_spat_adr]
    eas = [ea_acces, ea_desservi, ea_appartient, ea_contient]
    srcs = [_pad_e(ei[0], 0) for ei in eis]
    dstgs = [_pad_e(ei[1], 0) for ei in eis]
    dstss = [_pad_e(ei[1], N) for ei in eis]

    xs = {"adr": x_adresse, "bat": x_batiment, "par": x_parcelle}
    for l in range(2):
        new = {t: jnp.zeros((N, H), dtype=jnp.float32) for t in TYPES}
        for r, (s, d, ai) in enumerate(RELS):
            xl = xs[s] @ Wl[l, r]
            xr = xs[d] @ Wr[l, r]
            if ai is not None:
                ce = eas[ai] @ We[l, ai]
                ce = jnp.concatenate([ce, jnp.zeros((E_PAD - E, F), jnp.float32)])
            else:
                ce = None
            o = _edge_pass(xl, xr, ce, srcs[r], dstgs[r], dstss[r], att_w[l, r], eis[r][1])
            new[d] = new[d] + o + bias_w[l, r]
        xs = {t: jax.nn.relu(v) for t, v in new.items()}
    outs = [xs[t] @ lin_W[i] + lin_b[i] for i, t in enumerate(TYPES)]
    return jnp.stack(outs)


# speedup vs baseline: 4.5945x; 1.0439x over previous
import jax
import jax.numpy as jnp
from jax import lax
from jax.experimental import pallas as pl
from jax.experimental.pallas import tpu as pltpu
from jax.experimental.pallas import tpu_sc as plsc

N = 10000
E = 160000
F = 128
H = 128
OUT = 64
ED = 16
RELS = [("adr", "bat", 0), ("bat", "adr", 1), ("bat", "par", 2), ("par", "bat", 3), ("bat", "bat", None), ("par", "par", None), ("adr", "adr", None)]
TYPES = ["adr", "bat", "par"]

NC, NS, L = 2, 16, 16
NW = NC * NS
E_PAD = 163840
PER_W = E_PAD // NW
CHUNK = 64
NCH = PER_W // CHUNK
NP = 10112
STRIPE = NP // NS
KG = H // L


def _lane_sum_splat(v):
    dn = lax.GatherDimensionNumbers(
        offset_dims=(), collapsed_slice_dims=(0,), start_index_map=(0,))
    iota = lax.iota(jnp.int32, 16)
    for sh in (8, 4, 2, 1):
        idx = (iota + sh) & 15
        rot = lax.gather(v, idx[:, None], dn, slice_sizes=(1,),
                         mode=lax.GatherScatterMode.PROMISE_IN_BOUNDS)
        v = v + rot
    return v


def _edge_body(has_ce, refs):
    if has_ce:
        (xl_hbm, xr_hbm, ce_hbm, src_hbm, dstg_hbm, dsts_hbm, aw_hbm, zr_hbm,
         acc_out, ex_out,
         src_v, dstg_v, dsts_v, a0, a1, b0, b1, c0, ex_v, aw_v, acc_s,
         si1, si2, si3, sg1, sg2, sc, ss1, ss2) = refs
    else:
        (xl_hbm, xr_hbm, src_hbm, dstg_hbm, dsts_hbm, aw_hbm, zr_hbm,
         acc_out, ex_out,
         src_v, dstg_v, dsts_v, a0, a1, b0, b1, ex_v, aw_v, acc_s,
         si1, si2, si3, sg1, sg2, sc, ss1, ss2) = refs
        c0 = None
    av = [a0, a1]
    bv = [b0, b1]
    cid = lax.axis_index("c")
    sid = lax.axis_index("s")
    wid = sid * NC + cid
    base0 = wid * PER_W
    iota = lax.iota(jnp.int32, 16)
    zero16 = jnp.zeros((16,), jnp.float32)

    pltpu.sync_copy(zr_hbm, acc_s.at[pl.ds(sid * STRIPE, STRIPE)])
    plsc.subcore_barrier()

    pltpu.sync_copy(aw_hbm, aw_v)
    awk = [aw_v[pl.ds(16 * k, 16)] for k in range(KG)]

    def issue_isd(j, q4):
        base = base0 + j * CHUNK
        pltpu.async_copy(src_hbm.at[pl.ds(base, CHUNK)], src_v.at[q4], si1[q4])
        pltpu.async_copy(dstg_hbm.at[pl.ds(base, CHUNK)], dstg_v.at[q4], si2[q4])

    def issue_idst(j, p):
        base = base0 + j * CHUNK
        pltpu.async_copy(dsts_hbm.at[pl.ds(base, CHUNK)], dsts_v.at[p], si3[p])

    def wait_isd(q4):
        pltpu.make_async_copy(src_hbm.at[pl.ds(0, CHUNK)], src_v.at[q4], si1[q4]).wait()
        pltpu.make_async_copy(dstg_hbm.at[pl.ds(0, CHUNK)], dstg_v.at[q4], si2[q4]).wait()

    def issue_g(j, q4, p):
        pltpu.async_copy(xl_hbm.at[src_v.at[q4]], av[p], sg1[p])
        pltpu.async_copy(xr_hbm.at[dstg_v.at[q4]], bv[p], sg2[p])

    def issue_ce(j):
        if has_ce:
            base = base0 + j * CHUNK
            pltpu.async_copy(ce_hbm.at[pl.ds(base, CHUNK)], c0, sc)

    def wait_g(p):
        pltpu.make_async_copy(xl_hbm.at[src_v.at[0]], av[p], sg1[p]).wait()
        pltpu.make_async_copy(xr_hbm.at[dstg_v.at[0]], bv[p], sg2[p]).wait()
        if has_ce:
            pltpu.make_async_copy(ce_hbm.at[pl.ds(0, CHUNK)], c0, sc).wait()

    def issue_s(j, p):
        base = base0 + j * CHUNK
        pltpu.async_copy(av[p], acc_s.at[dsts_v.at[p]], ss1[p], add=True)
        pltpu.async_copy(ex_v.at[p], ex_out.at[pl.ds(base, CHUNK)], ss2[p])

    def drain_s(p):
        pltpu.make_async_copy(av[p], acc_s.at[dsts_v.at[p]], ss1[p]).wait()
        pltpu.make_async_copy(ex_v.at[p], ex_out.at[pl.ds(0, CHUNK)], ss2[p]).wait()

    def wait_idst(p):
        pltpu.make_async_copy(dsts_hbm.at[pl.ds(0, CHUNK)], dsts_v.at[p], si3[p]).wait()

    def compute(p):
        a_v = av[p]
        b_v = bv[p]
        c_v = c0

        def edge_step(e, _):
            aks = []
            acc = zero16
            for k in range(KG):
                ak = a_v[e, pl.ds(16 * k, 16)]
                aks.append(ak)
                m = ak + b_v[e, pl.ds(16 * k, 16)]
                if has_ce:
                    m = m + c_v[e, pl.ds(16 * k, 16)]
                lr = jnp.maximum(m, 0.2 * m)
                acc = acc + lr * awk[k]
            exv = jnp.exp(_lane_sum_splat(acc))
            for k in range(KG):
                a_v[e, pl.ds(16 * k, 16)] = aks[k] * exv
            e16 = (e >> 4) << 4
            win = ex_v[p, pl.ds(e16, 16)]
            ex_v[p, pl.ds(e16, 16)] = jnp.where(iota == (e & 15), exv, win)
            return 0

        lax.fori_loop(0, CHUNK, edge_step, 0)

    def proc(j, u, first=False, pre_i2=True, pre_n1=True):
        p = u & 1
        if pre_i2:
            issue_isd(j + 2, (u + 2) & 3)
        wait_g(p)
        compute(p)
        if pre_n1:
            issue_ce(j + 1)
        if not first:
            drain_s(1 - p)
        if pre_n1:
            issue_idst(j + 1, 1 - p)
        wait_idst(p)
        issue_s(j, p)
        if pre_n1:
            wait_isd((u + 1) & 3)
            issue_g(j + 1, (u + 1) & 3, 1 - p)

    issue_isd(0, 0)
    issue_isd(1, 1)
    issue_idst(0, 0)
    issue_ce(0)
    wait_isd(0)
    issue_g(0, 0, 0)

    proc(0, 0, first=True)

    def quad_shift(t, _):
        j = 1 + t * 4
        proc(j + 0, 1)
        proc(j + 1, 2)
        proc(j + 2, 3)
        proc(j + 3, 0)
        return 0

    lax.fori_loop(0, (NCH - 4) // 4, quad_shift, 0)
    proc(NCH - 3, 1)
    proc(NCH - 2, 2, pre_i2=False)
    proc(NCH - 1, 3, pre_i2=False, pre_n1=False)
    drain_s((NCH - 1) & 1)

    plsc.subcore_barrier()
    pltpu.sync_copy(acc_s.at[pl.ds(sid * STRIPE, STRIPE)],
                    acc_out.at[pl.ds(cid * NP + sid * STRIPE, STRIPE)])


def _make_edge_kernel(has_ce):
    mesh = plsc.VectorSubcoreMesh(core_axis_name="c", subcore_axis_name="s")
    scratch = [
        pltpu.VMEM((4, CHUNK), jnp.int32),
        pltpu.VMEM((4, CHUNK), jnp.int32),
        pltpu.VMEM((2, CHUNK), jnp.int32),
        pltpu.VMEM((CHUNK, F), jnp.float32),
        pltpu.VMEM((CHUNK, F), jnp.float32),
        pltpu.VMEM((CHUNK, F), jnp.float32),
        pltpu.VMEM((CHUNK, F), jnp.float32),
    ]
    if has_ce:
        scratch += [
            pltpu.VMEM((CHUNK, F), jnp.float32),
        ]
    scratch += [
        pltpu.VMEM((2, CHUNK), jnp.float32),
        pltpu.VMEM((F,), jnp.float32),
        pltpu.VMEM_SHARED((NP, F), jnp.float32),
        [pltpu.SemaphoreType.DMA] * 4,
        [pltpu.SemaphoreType.DMA] * 4,
        [pltpu.SemaphoreType.DMA] * 2,
        [pltpu.SemaphoreType.DMA] * 2,
        [pltpu.SemaphoreType.DMA] * 2,
        pltpu.SemaphoreType.DMA,
        [pltpu.SemaphoreType.DMA] * 2,
        [pltpu.SemaphoreType.DMA] * 2,
    ]

    def body(*refs):
        _edge_body(has_ce, refs)

    return pl.kernel(
        body,
        mesh=mesh,
        out_type=(
            jax.ShapeDtypeStruct((NC * NP, F), jnp.float32),
            jax.ShapeDtypeStruct((E_PAD,), jnp.float32),
        ),
        scratch_types=scratch,
    )


def _edge_pass(xl, xr, ce, srcp, dstgp, dstsp, aw, dst):
    zr = jnp.zeros((STRIPE, F), jnp.float32)
    k = _make_edge_kernel(ce is not None)
    if ce is not None:
        acc, ex = k(xl, xr, ce, srcp, dstgp, dstsp, aw, zr)
    else:
        acc, ex = k(xl, xr, srcp, dstgp, dstsp, aw, zr)
    acc = acc.reshape(NC, NP, F)
    num = acc[0, :N] + acc[1, :N]
    d = jax.ops.segment_sum(ex[:E], dst, num_segments=N)
    return num / (d + 1e-16)[:, None]


def _pad_e(ix, fill):
    return jnp.concatenate([ix, jnp.full((E_PAD - E,), fill, jnp.int32)])


def kernel(x_adresse, x_batiment, x_parcelle, ei_acces, ei_desservi, ei_appartient, ei_contient, ei_spat_bat, ei_spat_par, ei_spat_adr, ea_acces, ea_desservi, ea_appartient, ea_contient, Wl, Wr, att_w, bias_w, We, lin_W, lin_b):
    eis = [ei_acces, ei_desservi, ei_appartient, ei_contient, ei_spat_bat, ei_spat_par, ei_spat_adr]
    eas = [ea_acces, ea_desservi, ea_appartient, ea_contient]
    srcs = [_pad_e(ei[0], 0) for ei in eis]
    dstgs = [_pad_e(ei[1], 0) for ei in eis]
    dstss = [_pad_e(ei[1], N) for ei in eis]

    xs = {"adr": x_adresse, "bat": x_batiment, "par": x_parcelle}
    for l in range(2):
        new = {t: jnp.zeros((N, H), dtype=jnp.float32) for t in TYPES}
        for r, (s, d, ai) in enumerate(RELS):
            xl = xs[s] @ Wl[l, r]
            xr = xs[d] @ Wr[l, r]
            if ai is not None:
                ce = eas[ai] @ We[l, ai]
                ce = jnp.concatenate([ce, jnp.zeros((E_PAD - E, F), jnp.float32)])
            else:
                ce = None
            o = _edge_pass(xl, xr, ce, srcs[r], dstgs[r], dstss[r], att_w[l, r], eis[r][1])
            new[d] = new[d] + o + bias_w[l, r]
        xs = {t: jax.nn.relu(v) for t, v in new.items()}
    outs = [xs[t] @ lin_W[i] + lin_b[i] for i, t in enumerate(TYPES)]
    return jnp.stack(outs)


# speedup vs baseline: 5.1336x; 1.1173x over previous
import jax
import jax.numpy as jnp
from jax import lax
from jax.experimental import pallas as pl
from jax.experimental.pallas import tpu as pltpu
from jax.experimental.pallas import tpu_sc as plsc

N = 10000
E = 160000
F = 128
H = 128
OUT = 64
ED = 16
RELS = [("adr", "bat", 0), ("bat", "adr", 1), ("bat", "par", 2), ("par", "bat", 3), ("bat", "bat", None), ("par", "par", None), ("adr", "adr", None)]
TYPES = ["adr", "bat", "par"]

NC, NS, L = 2, 16, 16
NW = NC * NS
E_PAD = 163840
PER_W = E_PAD // NW
CHUNK = 64
NCH = PER_W // CHUNK
NP = 10112
STRIPE = NP // NS
KG = H // L


def _lane_sum_splat(v):
    dn = lax.GatherDimensionNumbers(
        offset_dims=(), collapsed_slice_dims=(0,), start_index_map=(0,))
    iota = lax.iota(jnp.int32, 16)
    for sh in (8, 4, 2, 1):
        idx = (iota + sh) & 15
        rot = lax.gather(v, idx[:, None], dn, slice_sizes=(1,),
                         mode=lax.GatherScatterMode.PROMISE_IN_BOUNDS)
        v = v + rot
    return v


def _edge_body(has_ce, refs):
    if has_ce:
        (xl_hbm, xr_hbm, ce_hbm, src_hbm, dstg_hbm, dsts_hbm, aw_hbm, zr_hbm,
         acc_out, ex_out,
         src_v, dstg_v, dsts_v, a0, a1, b0, b1, c0, ex_v, aw_v, acc_s,
         si1, si2, si3, sg1, sg2, sc, ss1, ss2) = refs
    else:
        (xl_hbm, xr_hbm, src_hbm, dstg_hbm, dsts_hbm, aw_hbm, zr_hbm,
         acc_out, ex_out,
         src_v, dstg_v, dsts_v, a0, a1, b0, b1, ex_v, aw_v, acc_s,
         si1, si2, si3, sg1, sg2, sc, ss1, ss2) = refs
        c0 = None
    av = [a0, a1]
    bv = [b0, b1]
    cid = lax.axis_index("c")
    sid = lax.axis_index("s")
    wid = sid * NC + cid
    base0 = wid * PER_W
    iota = lax.iota(jnp.int32, 16)
    zero16 = jnp.zeros((16,), jnp.float32)

    pltpu.sync_copy(zr_hbm, acc_s.at[pl.ds(sid * STRIPE, STRIPE)])
    plsc.subcore_barrier()

    pltpu.sync_copy(aw_hbm, aw_v)
    awk = [aw_v[pl.ds(16 * k, 16)] for k in range(KG)]

    def issue_isd(j, q4):
        base = base0 + j * CHUNK
        pltpu.async_copy(src_hbm.at[pl.ds(base, CHUNK)], src_v.at[q4], si1[q4])
        pltpu.async_copy(dstg_hbm.at[pl.ds(base, CHUNK)], dstg_v.at[q4], si2[q4])

    def issue_idst(j, p):
        base = base0 + j * CHUNK
        pltpu.async_copy(dsts_hbm.at[pl.ds(base, CHUNK)], dsts_v.at[p], si3[p])

    def wait_isd(q4):
        pltpu.make_async_copy(src_hbm.at[pl.ds(0, CHUNK)], src_v.at[q4], si1[q4]).wait()
        pltpu.make_async_copy(dstg_hbm.at[pl.ds(0, CHUNK)], dstg_v.at[q4], si2[q4]).wait()

    def issue_g(j, q4, p):
        pltpu.async_copy(xl_hbm.at[src_v.at[q4]], av[p], sg1[p])
        pltpu.async_copy(xr_hbm.at[dstg_v.at[q4]], bv[p], sg2[p])

    def issue_ce(j):
        if has_ce:
            base = base0 + j * CHUNK
            pltpu.async_copy(ce_hbm.at[pl.ds(base, CHUNK)], c0, sc)

    def wait_g(p):
        pltpu.make_async_copy(xl_hbm.at[src_v.at[0]], av[p], sg1[p]).wait()
        pltpu.make_async_copy(xr_hbm.at[dstg_v.at[0]], bv[p], sg2[p]).wait()
        if has_ce:
            pltpu.make_async_copy(ce_hbm.at[pl.ds(0, CHUNK)], c0, sc).wait()

    def issue_s(j, p):
        base = base0 + j * CHUNK
        pltpu.async_copy(av[p], acc_s.at[dsts_v.at[p]], ss1[p], add=True)
        pltpu.async_copy(ex_v.at[p], ex_out.at[pl.ds(base, CHUNK)], ss2[p])

    def drain_s(p):
        pltpu.make_async_copy(av[p], acc_s.at[dsts_v.at[p]], ss1[p]).wait()
        pltpu.make_async_copy(ex_v.at[p], ex_out.at[pl.ds(0, CHUNK)], ss2[p]).wait()

    def wait_idst(p):
        pltpu.make_async_copy(dsts_hbm.at[pl.ds(0, CHUNK)], dsts_v.at[p], si3[p]).wait()

    def compute(p):
        a_v = av[p]
        b_v = bv[p]
        c_v = c0

        def edge_step(i, _):
            e0 = i * 2
            e1 = e0 + 1
            aks = []
            exvs = []
            for e in (e0, e1):
                acc = zero16
                aa = []
                for k in range(KG):
                    ak = a_v[e, pl.ds(16 * k, 16)]
                    aa.append(ak)
                    m = ak + b_v[e, pl.ds(16 * k, 16)]
                    if has_ce:
                        m = m + c_v[e, pl.ds(16 * k, 16)]
                    lr = jnp.maximum(m, 0.2 * m)
                    acc = acc + lr * awk[k]
                aks.append(aa)
                exvs.append(jnp.exp(_lane_sum_splat(acc)))
            for t, e in enumerate((e0, e1)):
                for k in range(KG):
                    a_v[e, pl.ds(16 * k, 16)] = aks[t][k] * exvs[t]
            e16 = (e0 >> 4) << 4
            win = ex_v[p, pl.ds(e16, 16)]
            win = jnp.where(iota == (e0 & 15), exvs[0], win)
            ex_v[p, pl.ds(e16, 16)] = jnp.where(iota == (e1 & 15), exvs[1], win)
            return 0

        lax.fori_loop(0, CHUNK // 2, edge_step, 0)

    def proc(j, u, first=False, pre_i2=True, pre_n1=True):
        p = u & 1
        if pre_i2:
            issue_isd(j + 2, (u + 2) & 3)
        wait_g(p)
        compute(p)
        if pre_n1:
            issue_ce(j + 1)
        if not first:
            drain_s(1 - p)
        if pre_n1:
            issue_idst(j + 1, 1 - p)
        wait_idst(p)
        issue_s(j, p)
        if pre_n1:
            wait_isd((u + 1) & 3)
            issue_g(j + 1, (u + 1) & 3, 1 - p)

    issue_isd(0, 0)
    issue_isd(1, 1)
    issue_idst(0, 0)
    issue_ce(0)
    wait_isd(0)
    issue_g(0, 0, 0)

    proc(0, 0, first=True)

    def quad_shift(t, _):
        j = 1 + t * 4
        proc(j + 0, 1)
        proc(j + 1, 2)
        proc(j + 2, 3)
        proc(j + 3, 0)
        return 0

    lax.fori_loop(0, (NCH - 4) // 4, quad_shift, 0)
    proc(NCH - 3, 1)
    proc(NCH - 2, 2, pre_i2=False)
    proc(NCH - 1, 3, pre_i2=False, pre_n1=False)
    drain_s((NCH - 1) & 1)

    plsc.subcore_barrier()
    pltpu.sync_copy(acc_s.at[pl.ds(sid * STRIPE, STRIPE)],
                    acc_out.at[pl.ds(cid * NP + sid * STRIPE, STRIPE)])


def _make_edge_kernel(has_ce):
    mesh = plsc.VectorSubcoreMesh(core_axis_name="c", subcore_axis_name="s")
    scratch = [
        pltpu.VMEM((4, CHUNK), jnp.int32),
        pltpu.VMEM((4, CHUNK), jnp.int32),
        pltpu.VMEM((2, CHUNK), jnp.int32),
        pltpu.VMEM((CHUNK, F), jnp.float32),
        pltpu.VMEM((CHUNK, F), jnp.float32),
        pltpu.VMEM((CHUNK, F), jnp.float32),
        pltpu.VMEM((CHUNK, F), jnp.float32),
    ]
    if has_ce:
        scratch += [
            pltpu.VMEM((CHUNK, F), jnp.float32),
        ]
    scratch += [
        pltpu.VMEM((2, CHUNK), jnp.float32),
        pltpu.VMEM((F,), jnp.float32),
        pltpu.VMEM_SHARED((NP, F), jnp.float32),
        [pltpu.SemaphoreType.DMA] * 4,
        [pltpu.SemaphoreType.DMA] * 4,
        [pltpu.SemaphoreType.DMA] * 2,
        [pltpu.SemaphoreType.DMA] * 2,
        [pltpu.SemaphoreType.DMA] * 2,
        pltpu.SemaphoreType.DMA,
        [pltpu.SemaphoreType.DMA] * 2,
        [pltpu.SemaphoreType.DMA] * 2,
    ]

    def body(*refs):
        _edge_body(has_ce, refs)

    return pl.kernel(
        body,
        mesh=mesh,
        out_type=(
            jax.ShapeDtypeStruct((NC * NP, F), jnp.float32),
            jax.ShapeDtypeStruct((E_PAD,), jnp.float32),
        ),
        scratch_types=scratch,
    )


def _edge_pass(xl, xr, ce, srcp, dstgp, dstsp, aw, dst):
    zr = jnp.zeros((STRIPE, F), jnp.float32)
    k = _make_edge_kernel(ce is not None)
    if ce is not None:
        acc, ex = k(xl, xr, ce, srcp, dstgp, dstsp, aw, zr)
    else:
        acc, ex = k(xl, xr, srcp, dstgp, dstsp, aw, zr)
    acc = acc.reshape(NC, NP, F)
    num = acc[0, :N] + acc[1, :N]
    d = jax.ops.segment_sum(ex[:E], dst, num_segments=N)
    return num / (d + 1e-16)[:, None]


def _pad_e(ix, fill):
    return jnp.concatenate([ix, jnp.full((E_PAD - E,), fill, jnp.int32)])


def kernel(x_adresse, x_batiment, x_parcelle, ei_acces, ei_desservi, ei_appartient, ei_contient, ei_spat_bat, ei_spat_par, ei_spat_adr, ea_acces, ea_desservi, ea_appartient, ea_contient, Wl, Wr, att_w, bias_w, We, lin_W, lin_b):
    eis = [ei_acces, ei_desservi, ei_appartient, ei_contient, ei_spat_bat, ei_spat_par, ei_spat_adr]
    eas = [ea_acces, ea_desservi, ea_appartient, ea_contient]
    srcs = [_pad_e(ei[0], 0) for ei in eis]
    dstgs = [_pad_e(ei[1], 0) for ei in eis]
    dstss = [_pad_e(ei[1], N) for ei in eis]

    xs = {"adr": x_adresse, "bat": x_batiment, "par": x_parcelle}
    for l in range(2):
        new = {t: jnp.zeros((N, H), dtype=jnp.float32) for t in TYPES}
        for r, (s, d, ai) in enumerate(RELS):
            xl = xs[s] @ Wl[l, r]
            xr = xs[d] @ Wr[l, r]
            if ai is not None:
                ce = eas[ai] @ We[l, ai]
                ce = jnp.concatenate([ce, jnp.zeros((E_PAD - E, F), jnp.float32)])
            else:
                ce = None
            o = _edge_pass(xl, xr, ce, srcs[r], dstgs[r], dstss[r], att_w[l, r], eis[r][1])
            new[d] = new[d] + o + bias_w[l, r]
        xs = {t: jax.nn.relu(v) for t, v in new.items()}
    outs = [xs[t] @ lin_W[i] + lin_b[i] for i, t in enumerate(TYPES)]
    return jnp.stack(outs)
